# baseline (speedup 1.0000x reference)
import functools

import jax
import jax.numpy as jnp
from jax import lax
from jax.experimental import pallas as pl
from jax.experimental.pallas import tpu as pltpu
from jax.experimental.pallas import tpu_sc as plsc

_N = 1048576
_C = 19
_K = 2048
_STRIDE = 48
_NB2 = _K * _STRIDE
_NW = 32
_M = _N * _C
_PER_W = _M // _NW
_CHUNK = 8192
_NCHUNK = _PER_W // _CHUNK
_BLK = 4096


_W = 128 * _C
_NR = _N // 128
_RB = 512


def _bucketize_body(x_ref, t_ref, ids_ref):
    x = x_ref[...]
    t = t_ref[...]
    ex = jnp.exp(jnp.clip(x, -60.0, 60.0))
    jj = lax.broadcasted_iota(jnp.int32, (_W, 128), 0) // _C
    rr = lax.broadcasted_iota(jnp.int32, (_W, 128), 1)
    sel = (jj == rr).astype(jnp.bfloat16)
    jj2 = lax.broadcasted_iota(jnp.int32, (128, _W), 1) // _C
    rr2 = lax.broadcasted_iota(jnp.int32, (128, _W), 0)
    selt = (jj2 == rr2).astype(jnp.bfloat16)
    s = jnp.dot(ex.astype(jnp.bfloat16), sel,
                preferred_element_type=jnp.float32)
    r = _K / s
    rexp = jnp.dot(r.astype(jnp.bfloat16), selt,
                   preferred_element_type=jnp.float32)
    texp = jnp.dot(t.astype(jnp.bfloat16), selt,
                   preferred_element_type=jnp.float32)
    cl = lax.broadcasted_iota(jnp.int32, x.shape, 1) % _C
    fg = cl.astype(jnp.float32) == texp
    b = jnp.clip((ex * rexp).astype(jnp.int32), 0, _K - 1)
    bv = jnp.where(fg, (_K - 1) - b, b)
    ids_ref[...] = bv * _STRIDE + cl + jnp.where(fg, _C, 0)


_UNROLL = 8


def _sc_hist_body(ids_hbm, out_hbm, buf0, buf1, hist, sem0, sem1):
    wid = lax.axis_index("s") * 2 + lax.axis_index("c")
    base = wid * _PER_W
    zeros16 = jnp.zeros((16,), jnp.float32)
    ones16 = jnp.ones((16,), jnp.float32)

    def zero_body(j, carry):
        off = pl.multiple_of(j * (16 * _UNROLL), 16 * _UNROLL)
        for u in range(_UNROLL):
            hist[pl.ds(off + u * 16, 16)] = zeros16
        return carry

    lax.fori_loop(0, _NB2 // (16 * _UNROLL), zero_body, 0)

    def copy_in(k, buf, sem):
        return pltpu.make_async_copy(
            ids_hbm.at[pl.ds(base + k * _CHUNK, _CHUNK)], buf, sem)

    def scatter_chunk(buf):
        def scat(j, c2):
            off = pl.multiple_of(j * (16 * _UNROLL), 16 * _UNROLL)
            for u in range(_UNROLL):
                idx = buf[pl.ds(off + u * 16, 16)]
                plsc.addupdate_scatter(hist, [idx], ones16)
            return c2

        lax.fori_loop(0, _CHUNK // (16 * _UNROLL), scat, 0)

    copy_in(0, buf0, sem0).start()

    def chunk_pair(k2, carry):
        k = k2 * 2
        copy_in(0, buf0, sem0).wait()
        copy_in(k + 1, buf1, sem1).start()
        scatter_chunk(buf0)
        copy_in(0, buf1, sem1).wait()

        @pl.when(k + 2 < _NCHUNK)
        def _():
            copy_in(k + 2, buf0, sem0).start()

        scatter_chunk(buf1)
        return carry

    lax.fori_loop(0, _NCHUNK // 2, chunk_pair, 0)
    pltpu.sync_copy(hist, out_hbm.at[wid])


@functools.cache
def _sc_hist():
    return pl.kernel(
        _sc_hist_body,
        mesh=plsc.VectorSubcoreMesh(core_axis_name="c", subcore_axis_name="s"),
        out_type=jax.ShapeDtypeStruct((_NW, _NB2), jnp.float32),
        scratch_types=[
            pltpu.VMEM((_CHUNK,), jnp.int32),
            pltpu.VMEM((_CHUNK,), jnp.int32),
            pltpu.VMEM((_NB2,), jnp.float32),
            pltpu.SemaphoreType.DMA,
            pltpu.SemaphoreType.DMA,
        ],
        compiler_params=pltpu.CompilerParams(needs_layout_passes=False),
    )


def _reduce_body(h_ref, o_ref):
    h = h_ref[...]
    hs = jnp.sum(h, axis=0)
    row = lax.broadcasted_iota(jnp.int32, (_K, _K), 0)
    col = lax.broadcasted_iota(jnp.int32, (_K, _K), 1)
    u = (col >= row).astype(jnp.float32)
    s = jnp.dot(u, hs, preferred_element_type=jnp.float32)
    b = s[:, :_C]
    a = s[:, _C:2 * _C]
    g = a[0:1, :]
    jmat = (a + b) / jnp.maximum(g + b, 1.0)
    jsum = jnp.sum(jmat, axis=0, keepdims=True)
    loss_c = (jsum - 0.5 * jmat[0:1, :]) * (1.0 / _K)
    present = g > 0.0
    acc = jnp.sum(jnp.where(present, loss_c, 0.0))
    cnt = jnp.sum(jnp.where(present, 1.0, 0.0))
    o_ref[...] = (acc / cnt).reshape(1, 1)


def kernel(outputs, targets):
    ids = pl.pallas_call(
        _bucketize_body,
        grid=(_NR // _RB,),
        in_specs=[
            pl.BlockSpec((_RB, _W), lambda i: (i, 0)),
            pl.BlockSpec((_RB, 128), lambda i: (i, 0)),
        ],
        out_specs=pl.BlockSpec((_RB, _W), lambda i: (i, 0)),
        out_shape=jax.ShapeDtypeStruct((_NR, _W), jnp.int32),
    )(outputs.reshape(_NR, _W), targets.reshape(_NR, 128))
    hists = _sc_hist()(ids.reshape(_M))
    out = pl.pallas_call(
        _reduce_body,
        out_shape=jax.ShapeDtypeStruct((1, 1), jnp.float32),
    )(hists.reshape(_NW, _K, _STRIDE))
    return out.reshape(())


# speedup vs baseline: 1.1096x; 1.1096x over previous
import functools

import jax
import jax.numpy as jnp
from jax import lax
from jax.experimental import pallas as pl
from jax.experimental.pallas import tpu as pltpu
from jax.experimental.pallas import tpu_sc as plsc

_N = 1048576
_C = 19
_K = 2048
_NB2 = 2 * _C * _K
_NW = 32
_M = _N * _C
_RPW = 8192 // _NW
_RCH = 8
_NCHUNK = _RPW // _RCH
_BLK = 4096


_W = 128 * _C
_NR = _N // 128
_RB = 512


def _bucketize_body(x_ref, t_ref, ids_ref):
    x = x_ref[...]
    t = t_ref[...]
    ex = jnp.exp(jnp.clip(x, -60.0, 60.0))
    jj = lax.broadcasted_iota(jnp.int32, (_W, 128), 0) // _C
    rr = lax.broadcasted_iota(jnp.int32, (_W, 128), 1)
    sel = (jj == rr).astype(jnp.bfloat16)
    jj2 = lax.broadcasted_iota(jnp.int32, (128, _W), 1) // _C
    rr2 = lax.broadcasted_iota(jnp.int32, (128, _W), 0)
    selt = (jj2 == rr2).astype(jnp.bfloat16)
    s = jnp.dot(ex.astype(jnp.bfloat16), sel,
                preferred_element_type=jnp.float32)
    r = _K / s
    rexp = jnp.dot(r.astype(jnp.bfloat16), selt,
                   preferred_element_type=jnp.float32)
    texp = jnp.dot(t.astype(jnp.bfloat16), selt,
                   preferred_element_type=jnp.float32)
    cl = lax.broadcasted_iota(jnp.int32, x.shape, 1) % _C
    fg = cl.astype(jnp.float32) == texp
    b = jnp.clip((ex * rexp).astype(jnp.int32), 0, _K - 1)
    ids_ref[...] = (jnp.where(fg, _C * _K, 0) + cl * _K
                    + jnp.where(fg, (_K - 1) - b, b))


_UNROLL = 8


def _sc_hist_body(ids_hbm, out_hbm, buf0, buf1, hist, sem0, sem1):
    wid = lax.axis_index("s") * 2 + lax.axis_index("c")
    rbase = wid * _RPW
    zeros16 = jnp.zeros((16,), jnp.float32)
    ones16 = jnp.ones((16,), jnp.float32)

    def zero_body(j, carry):
        off = pl.multiple_of(j * (16 * _UNROLL), 16 * _UNROLL)
        for u in range(_UNROLL):
            hist[pl.ds(off + u * 16, 16)] = zeros16
        return carry

    lax.fori_loop(0, _NB2 // (16 * _UNROLL), zero_body, 0)

    def copy_in(k, buf, sem):
        return pltpu.make_async_copy(
            ids_hbm.at[pl.ds(rbase + k * _RCH, _RCH), :], buf, sem)

    def scatter_chunk(buf):
        def scat(j, c2):
            off = pl.multiple_of(j * 16, 16)
            for r in range(_RCH):
                idx = buf[r, pl.ds(off, 16)]
                plsc.addupdate_scatter(hist, [idx], ones16)
            return c2

        lax.fori_loop(0, _W // 16, scat, 0)

    copy_in(0, buf0, sem0).start()

    def chunk_pair(k2, carry):
        k = k2 * 2
        copy_in(0, buf0, sem0).wait()
        copy_in(k + 1, buf1, sem1).start()
        scatter_chunk(buf0)
        copy_in(0, buf1, sem1).wait()

        @pl.when(k + 2 < _NCHUNK)
        def _():
            copy_in(k + 2, buf0, sem0).start()

        scatter_chunk(buf1)
        return carry

    lax.fori_loop(0, _NCHUNK // 2, chunk_pair, 0)
    pltpu.sync_copy(hist, out_hbm.at[pl.ds(wid * _NB2, _NB2)])


@functools.cache
def _sc_hist():
    return pl.kernel(
        _sc_hist_body,
        mesh=plsc.VectorSubcoreMesh(core_axis_name="c", subcore_axis_name="s"),
        out_type=jax.ShapeDtypeStruct((_NW * _NB2,), jnp.float32),
        scratch_types=[
            pltpu.VMEM((_RCH, _W), jnp.int32),
            pltpu.VMEM((_RCH, _W), jnp.int32),
            pltpu.VMEM((_NB2,), jnp.float32),
            pltpu.SemaphoreType.DMA,
            pltpu.SemaphoreType.DMA,
        ],
        compiler_params=pltpu.CompilerParams(
            needs_layout_passes=False, use_tc_tiling_on_sc=True),
    )


def _reduce_body(h_ref, o_ref):
    h = h_ref[...]
    hs = jnp.sum(h, axis=0)
    h0 = hs[:_C]
    h1 = hs[_C:]
    row = lax.broadcasted_iota(jnp.int32, (_K, _K), 0)
    col = lax.broadcasted_iota(jnp.int32, (_K, _K), 1)
    u = (row >= col).astype(jnp.float32)
    a = jnp.dot(h1, u, preferred_element_type=jnp.float32)
    b = jnp.dot(h0, u, preferred_element_type=jnp.float32)
    g = a[:, 0:1]
    jmat = (a + b) / jnp.maximum(g + b, 1.0)
    jsum = jnp.sum(jmat, axis=1, keepdims=True)
    loss_c = (jsum - 0.5 * jmat[:, 0:1]) * (1.0 / _K)
    present = g > 0.0
    acc = jnp.sum(jnp.where(present, loss_c, 0.0))
    cnt = jnp.sum(jnp.where(present, 1.0, 0.0))
    o_ref[...] = (acc / cnt).reshape(1, 1)


def kernel(outputs, targets):
    ids = pl.pallas_call(
        _bucketize_body,
        grid=(_NR // _RB,),
        in_specs=[
            pl.BlockSpec((_RB, _W), lambda i: (i, 0)),
            pl.BlockSpec((_RB, 128), lambda i: (i, 0)),
        ],
        out_specs=pl.BlockSpec((_RB, _W), lambda i: (i, 0)),
        out_shape=jax.ShapeDtypeStruct((_NR, _W), jnp.int32),
    )(outputs.reshape(_NR, _W), targets.reshape(_NR, 128))
    hists = _sc_hist()(ids)
    out = pl.pallas_call(
        _reduce_body,
        out_shape=jax.ShapeDtypeStruct((1, 1), jnp.float32),
    )(hists.reshape(_NW, 2 * _C, _K))
    return out.reshape(())
